# baseline (speedup 1.0000x reference)
import functools

import jax
import jax.numpy as jnp
from jax import lax
from jax.experimental import pallas as pl
from jax.experimental.pallas import tpu as pltpu
from jax.experimental.pallas import tpu_sc as plsc

N_NODES = 10000
N_EDGES = 320000
D = 128
N_GRAPHS = 64
D_OUT = 100

NC = 2
NS = 16
NW = NC * NS
EW = N_EDGES // NW
K = 80
CH = 624
KA = 128
EWP = 10240
NCH = EWP // KA
ACC_ROWS = N_NODES + 16
TAIL = N_NODES - NS * CH

R = 2000
GRID = N_NODES // R


@functools.cache
def _sc_kernels():
    mesh = plsc.VectorSubcoreMesh(core_axis_name="c", subcore_axis_name="s",
                                  num_cores=NC, num_subcores=NS)

    @functools.partial(
        pl.kernel,
        out_type=jax.ShapeDtypeStruct((NW, N_NODES), jnp.float32),
        mesh=mesh,
        compiler_params=pltpu.CompilerParams(needs_layout_passes=False),
        scratch_types=[
            pltpu.VMEM((K,), jnp.int32),
            pltpu.VMEM((16,), jnp.float32),
            pltpu.VMEM((N_NODES,), jnp.float32),
        ],
    )
    def deg_kernel(dst_hbm, out_hbm, didx, ones_v, tab):
        c = lax.axis_index("c")
        s = lax.axis_index("s")
        wid = s * NC + c
        ones_v[...] = jnp.full((16,), 1.0, jnp.float32)
        zv = ones_v[...] * 0.0

        def zbody(i, carry):
            tab[pl.ds(i * 16, 16)] = zv
            return carry

        lax.fori_loop(0, N_NODES // 16, zbody, 0)
        base = wid * EW

        def body(i, carry):
            pltpu.sync_copy(dst_hbm.at[pl.ds(base + i * K, K)], didx)
            ov = ones_v[...]
            for j in range(K // 16):
                dvec = didx[pl.ds(j * 16, 16)]
                plsc.addupdate_scatter(tab, [dvec], ov)
            return carry

        lax.fori_loop(0, EW // K, body, 0)
        pltpu.sync_copy(tab, out_hbm.at[wid])

    @functools.partial(
        pl.kernel,
        out_type=jax.ShapeDtypeStruct((NC, N_NODES, D), jnp.float32),
        mesh=mesh,
        scratch_types=[
            pltpu.VMEM((2, KA), jnp.int32),
            pltpu.VMEM((2, KA), jnp.int32),
            pltpu.VMEM((2, KA), jnp.int32),
            pltpu.VMEM((2, KA), jnp.int32),
            pltpu.VMEM((KA, D), jnp.float32),
            pltpu.VMEM((KA, D), jnp.float32),
            pltpu.VMEM_SHARED((ACC_ROWS, D), jnp.float32),
            pltpu.SemaphoreType.DMA,
            pltpu.SemaphoreType.DMA,
            pltpu.SemaphoreType.DMA,
            pltpu.SemaphoreType.DMA,
            pltpu.SemaphoreType.DMA,
            pltpu.SemaphoreType.DMA,
        ],
    )
    def agg_kernel(hw_hbm, eidx_hbm, zeros_hbm, out_hbm,
                   b0, b1, b2, b3, rows_a, rows_b, acc,
                   sem_a, sem_b, is0, is1, is2, is3):
        c = lax.axis_index("c")
        s = lax.axis_index("s")
        wid = s * NC + c
        pltpu.sync_copy(zeros_hbm.at[pl.ds(0, CH)], acc.at[pl.ds(s * CH, CH)])

        @pl.when(s == NS - 1)
        def _():
            pltpu.sync_copy(zeros_hbm.at[pl.ds(0, TAIL)],
                            acc.at[pl.ds(NS * CH, TAIL)])

        plsc.subcore_barrier()

        dummy = hw_hbm.at[pl.ds(0, KA)]
        idummy = eidx_hbm.at[0, 0]
        ebase = eidx_hbm.at[wid]

        pltpu.sync_copy(ebase.at[0], b0)
        pltpu.async_copy(hw_hbm.at[b0.at[0]], rows_a, sem_a)
        pltpu.async_copy(ebase.at[1], b1, is1)
        pltpu.async_copy(ebase.at[2], b2, is2)
        pltpu.async_copy(ebase.at[3], b3, is3)

        def body(g, carry):
            n = 4 * g
            pltpu.make_async_copy(idummy, b1, is1).wait()
            pltpu.async_copy(hw_hbm.at[b1.at[0]], rows_b, sem_b)
            pltpu.make_async_copy(dummy, rows_a, sem_a).wait()
            pltpu.sync_copy(rows_a, acc.at[b0.at[1]], add=True)
            pltpu.async_copy(ebase.at[n + 4], b0, is0)
            pltpu.make_async_copy(idummy, b2, is2).wait()
            pltpu.async_copy(hw_hbm.at[b2.at[0]], rows_a, sem_a)
            pltpu.make_async_copy(dummy, rows_b, sem_b).wait()
            pltpu.sync_copy(rows_b, acc.at[b1.at[1]], add=True)
            pltpu.async_copy(ebase.at[n + 5], b1, is1)
            pltpu.make_async_copy(idummy, b3, is3).wait()
            pltpu.async_copy(hw_hbm.at[b3.at[0]], rows_b, sem_b)
            pltpu.make_async_copy(dummy, rows_a, sem_a).wait()
            pltpu.sync_copy(rows_a, acc.at[b2.at[1]], add=True)
            pltpu.async_copy(ebase.at[n + 6], b2, is2)
            pltpu.make_async_copy(idummy, b0, is0).wait()
            pltpu.async_copy(hw_hbm.at[b0.at[0]], rows_a, sem_a)
            pltpu.make_async_copy(dummy, rows_b, sem_b).wait()
            pltpu.sync_copy(rows_b, acc.at[b3.at[1]], add=True)
            pltpu.async_copy(ebase.at[n + 7], b3, is3)
            return carry

        lax.fori_loop(0, NCH // 4 - 1, body, 0)
        pltpu.make_async_copy(idummy, b1, is1).wait()
        pltpu.async_copy(hw_hbm.at[b1.at[0]], rows_b, sem_b)
        pltpu.make_async_copy(dummy, rows_a, sem_a).wait()
        pltpu.sync_copy(rows_a, acc.at[b0.at[1]], add=True)
        pltpu.make_async_copy(idummy, b2, is2).wait()
        pltpu.async_copy(hw_hbm.at[b2.at[0]], rows_a, sem_a)
        pltpu.make_async_copy(dummy, rows_b, sem_b).wait()
        pltpu.sync_copy(rows_b, acc.at[b1.at[1]], add=True)
        pltpu.make_async_copy(idummy, b3, is3).wait()
        pltpu.async_copy(hw_hbm.at[b3.at[0]], rows_b, sem_b)
        pltpu.make_async_copy(dummy, rows_a, sem_a).wait()
        pltpu.sync_copy(rows_a, acc.at[b2.at[1]], add=True)
        pltpu.make_async_copy(dummy, rows_b, sem_b).wait()
        pltpu.sync_copy(rows_b, acc.at[b3.at[1]], add=True)

        plsc.subcore_barrier()
        pltpu.sync_copy(acc.at[pl.ds(s * CH, CH)],
                        out_hbm.at[c, pl.ds(s * CH, CH)])

        @pl.when(s == NS - 1)
        def _():
            pltpu.sync_copy(acc.at[pl.ds(NS * CH, TAIL)],
                            out_hbm.at[c, pl.ds(NS * CH, TAIL)])

    return deg_kernel, agg_kernel


def _dinv_block(degw_ref):
    ones_col = jnp.ones((NW, 1), jnp.float32)
    deg = jnp.dot(degw_ref[...], ones_col,
                  preferred_element_type=jnp.float32) + 1.0
    return lax.rsqrt(deg)


def _enc_body(x_ref, degw_ref, W1_ref, b1_ref, W2_ref, b2_ref, Wc1_ref, out_ref):
    h = jnp.maximum(jnp.dot(x_ref[...], W1_ref[...],
                            preferred_element_type=jnp.float32) + b1_ref[...], 0.0)
    h = jnp.dot(h, W2_ref[...], preferred_element_type=jnp.float32) + b2_ref[...]
    hw = jnp.dot(h, Wc1_ref[...], preferred_element_type=jnp.float32)
    out_ref[...] = hw * _dinv_block(degw_ref)


def _mid_body(accp_ref, hws_ref, degw_ref, bc1_ref, Wc2_ref, out_ref):
    dinv = _dinv_block(degw_ref)
    out1 = (accp_ref[0] + accp_ref[1] + hws_ref[...]) * dinv + bc1_ref[...]
    out_ref[...] = jnp.dot(out1, Wc2_ref[...],
                           preferred_element_type=jnp.float32) * dinv


def _fin_body(accp_ref, hws_ref, degw_ref, bc2_ref, batch_ref, Wcls_ref, bcls_ref,
              h_ref, logits_ref, seg_acc, cnt_acc):
    i = pl.program_id(0)
    dinv = _dinv_block(degw_ref)
    h = (accp_ref[0] + accp_ref[1] + hws_ref[...]) * dinv + bc2_ref[...]
    h_ref[...] = h
    gids = lax.broadcasted_iota(jnp.int32, (N_GRAPHS, R), 0)
    onehot = (batch_ref[0] == gids).astype(jnp.float32)
    part = jnp.dot(onehot, h, preferred_element_type=jnp.float32)
    cnt = jnp.broadcast_to(jnp.sum(onehot, axis=1, keepdims=True), (N_GRAPHS, D))

    @pl.when(i == 0)
    def _():
        seg_acc[...] = part
        cnt_acc[...] = cnt

    @pl.when(i > 0)
    def _():
        seg_acc[...] += part
        cnt_acc[...] += cnt

    @pl.when(i == pl.num_programs(0) - 1)
    def _():
        hg = seg_acc[...] / jnp.maximum(cnt_acc[...], 1.0)
        logits_ref[...] = jnp.dot(hg, Wcls_ref[...],
                                  preferred_element_type=jnp.float32) + bcls_ref[...]


def _row_spec(width):
    return pl.BlockSpec((R, width), lambda i: (i, 0))


def _degw_spec():
    return pl.BlockSpec((R, NW), lambda i: (i, 0))


def _part_spec(width):
    return pl.BlockSpec((NC, R, width), lambda i: (0, i, 0))


def _full_spec(shape):
    nd = len(shape)
    return pl.BlockSpec(shape, lambda i, _nd=nd: (0,) * _nd)


_enc_call = pl.pallas_call(
    _enc_body,
    grid=(GRID,),
    in_specs=[
        _row_spec(D), _degw_spec(),
        _full_spec((D, D)), _full_spec((1, D)),
        _full_spec((D, D)), _full_spec((1, D)),
        _full_spec((D, D)),
    ],
    out_specs=_row_spec(D),
    out_shape=jax.ShapeDtypeStruct((N_NODES, D), jnp.float32),
)

_mid_call = pl.pallas_call(
    _mid_body,
    grid=(GRID,),
    in_specs=[
        _part_spec(D), _row_spec(D), _degw_spec(),
        _full_spec((1, D)), _full_spec((D, D)),
    ],
    out_specs=_row_spec(D),
    out_shape=jax.ShapeDtypeStruct((N_NODES, D), jnp.float32),
)

_fin_call = pl.pallas_call(
    _fin_body,
    grid=(GRID,),
    in_specs=[
        _part_spec(D), _row_spec(D), _degw_spec(),
        _full_spec((1, D)),
        pl.BlockSpec((1, 1, R), lambda i: (i, 0, 0)),
        _full_spec((D, D)), _full_spec((1, D)),
    ],
    out_specs=[_row_spec(D), _full_spec((N_GRAPHS, D))],
    out_shape=[
        jax.ShapeDtypeStruct((N_NODES, D), jnp.float32),
        jax.ShapeDtypeStruct((N_GRAPHS, D), jnp.float32),
    ],
    scratch_shapes=[
        pltpu.VMEM((N_GRAPHS, D), jnp.float32),
        pltpu.VMEM((N_GRAPHS, D), jnp.float32),
    ],
)


def kernel(x, edge_index, batch, W1, b1, W2, b2, Wc1, bc1, Wc2, bc2, Wcls, bcls):
    src = edge_index[0]
    dst = edge_index[1]
    srcp = jnp.pad(src.reshape(NW, EW), ((0, 0), (0, EWP - EW)))
    dstp = jnp.pad(dst.reshape(NW, EW), ((0, 0), (0, EWP - EW)),
                   constant_values=N_NODES)
    eidx = jnp.stack([srcp.reshape(NW, NCH, KA), dstp.reshape(NW, NCH, KA)], axis=2)
    zeros128 = jnp.zeros((CH, D), jnp.float32)

    _deg_kernel, _agg_kernel = _sc_kernels()
    degw = _deg_kernel(dst).T

    hw1s = _enc_call(x, degw, W1, b1.reshape(1, D), W2, b2.reshape(1, D), Wc1)
    acc1 = _agg_kernel(hw1s, eidx, zeros128)
    hw2s = _mid_call(acc1, hw1s, degw, bc1.reshape(1, D), Wc2)
    acc2 = _agg_kernel(hw2s, eidx, zeros128)

    Wcls_p = jnp.pad(Wcls, ((0, 0), (0, D - D_OUT)))
    bcls_p = jnp.pad(bcls, (0, D - D_OUT)).reshape(1, D)
    batch_r = batch.reshape(GRID, 1, R)
    h_out, logits_p = _fin_call(acc2, hw2s, degw, bc2.reshape(1, D),
                                batch_r, Wcls_p, bcls_p)
    return (logits_p[:, :D_OUT], h_out)


# speedup vs baseline: 2.1416x; 2.1416x over previous
import functools

import jax
import jax.numpy as jnp
from jax import lax
from jax.experimental import pallas as pl
from jax.experimental.pallas import tpu as pltpu
from jax.experimental.pallas import tpu_sc as plsc

N_NODES = 10000
N_EDGES = 320000
D = 128
N_GRAPHS = 64
D_OUT = 100

NC = 2
NS = 16
NW = NC * NS
EW = N_EDGES // NW
K = 80
CH = 624
TAIL = N_NODES - NS * CH

R = 2000
GRID = N_NODES // R


@functools.cache
def _sc_kernels():
    mesh = plsc.VectorSubcoreMesh(core_axis_name="c", subcore_axis_name="s",
                                  num_cores=NC, num_subcores=NS)

    @functools.partial(
        pl.kernel,
        out_type=jax.ShapeDtypeStruct((NW, N_NODES), jnp.float32),
        mesh=mesh,
        compiler_params=pltpu.CompilerParams(needs_layout_passes=False),
        scratch_types=[
            pltpu.VMEM((K,), jnp.int32),
            pltpu.VMEM((16,), jnp.float32),
            pltpu.VMEM((N_NODES,), jnp.float32),
        ],
    )
    def deg_kernel(dst_hbm, out_hbm, didx, ones_v, tab):
        c = lax.axis_index("c")
        s = lax.axis_index("s")
        wid = s * NC + c
        ones_v[...] = jnp.full((16,), 1.0, jnp.float32)
        zv = ones_v[...] * 0.0

        def zbody(i, carry):
            tab[pl.ds(i * 16, 16)] = zv
            return carry

        lax.fori_loop(0, N_NODES // 16, zbody, 0)
        base = wid * EW

        def body(i, carry):
            pltpu.sync_copy(dst_hbm.at[pl.ds(base + i * K, K)], didx)
            ov = ones_v[...]
            for j in range(K // 16):
                dvec = didx[pl.ds(j * 16, 16)]
                plsc.addupdate_scatter(tab, [dvec], ov)
            return carry

        lax.fori_loop(0, EW // K, body, 0)
        pltpu.sync_copy(tab, out_hbm.at[wid])

    NCHUNK = EW // K

    @functools.partial(
        pl.kernel,
        out_type=jax.ShapeDtypeStruct((NC, N_NODES, D), jnp.float32),
        mesh=mesh,
        scratch_types=[
            pltpu.VMEM((K,), jnp.int32),
            pltpu.VMEM((K,), jnp.int32),
            pltpu.VMEM((K,), jnp.int32),
            pltpu.VMEM((K,), jnp.int32),
            pltpu.VMEM((K, D), jnp.float32),
            pltpu.VMEM((K, D), jnp.float32),
            pltpu.VMEM_SHARED((N_NODES, D), jnp.float32),
            pltpu.SemaphoreType.DMA,
            pltpu.SemaphoreType.DMA,
            pltpu.SemaphoreType.DMA,
            pltpu.SemaphoreType.DMA,
        ],
    )
    def agg_kernel(hw_hbm, src_hbm, dst_hbm, zeros_hbm, out_hbm,
                   sidx_a, didx_a, sidx_b, didx_b, rows_a, rows_b, acc,
                   sem_a, sem_b, isem_a, isem_b):
        c = lax.axis_index("c")
        s = lax.axis_index("s")
        wid = s * NC + c
        pltpu.sync_copy(zeros_hbm.at[pl.ds(0, CH)], acc.at[pl.ds(s * CH, CH)])

        @pl.when(s == NS - 1)
        def _():
            pltpu.sync_copy(zeros_hbm.at[pl.ds(0, TAIL)],
                            acc.at[pl.ds(NS * CH, TAIL)])

        plsc.subcore_barrier()

        sbase = src_hbm.at[wid]
        dbase = dst_hbm.at[wid]
        dummy = hw_hbm.at[pl.ds(0, K)]
        idummy = sbase.at[0]

        pltpu.sync_copy(sbase.at[0], sidx_a)
        pltpu.sync_copy(dbase.at[0], didx_a)
        pltpu.async_copy(hw_hbm.at[sidx_a], rows_a, sem_a)
        pltpu.async_copy(sbase.at[1], sidx_b, isem_b)
        pltpu.async_copy(dbase.at[1], didx_b, isem_b)

        def body(i, carry):
            n = 2 * i
            pltpu.make_async_copy(idummy, sidx_b, isem_b).wait()
            pltpu.make_async_copy(idummy, didx_b, isem_b).wait()
            pltpu.async_copy(hw_hbm.at[sidx_b], rows_b, sem_b)
            pltpu.make_async_copy(dummy, rows_a, sem_a).wait()
            pltpu.sync_copy(rows_a, acc.at[didx_a], add=True)
            pltpu.async_copy(sbase.at[n + 2], sidx_a, isem_a)
            pltpu.async_copy(dbase.at[n + 2], didx_a, isem_a)
            pltpu.make_async_copy(idummy, sidx_a, isem_a).wait()
            pltpu.make_async_copy(idummy, didx_a, isem_a).wait()
            pltpu.async_copy(hw_hbm.at[sidx_a], rows_a, sem_a)
            pltpu.make_async_copy(dummy, rows_b, sem_b).wait()
            pltpu.sync_copy(rows_b, acc.at[didx_b], add=True)
            nn = jnp.minimum(n + 3, NCHUNK - 1)
            pltpu.async_copy(sbase.at[nn], sidx_b, isem_b)
            pltpu.async_copy(dbase.at[nn], didx_b, isem_b)
            return carry

        lax.fori_loop(0, (NCHUNK - 1) // 2, body, 0)
        pltpu.make_async_copy(idummy, sidx_b, isem_b).wait()
        pltpu.make_async_copy(idummy, didx_b, isem_b).wait()
        pltpu.make_async_copy(dummy, rows_a, sem_a).wait()
        pltpu.sync_copy(rows_a, acc.at[didx_a], add=True)

        plsc.subcore_barrier()
        pltpu.sync_copy(acc.at[pl.ds(s * CH, CH)],
                        out_hbm.at[c, pl.ds(s * CH, CH)])

        @pl.when(s == NS - 1)
        def _():
            pltpu.sync_copy(acc.at[pl.ds(NS * CH, TAIL)],
                            out_hbm.at[c, pl.ds(NS * CH, TAIL)])

    return deg_kernel, agg_kernel


def _dinv_block(degw_ref):
    ones_col = jnp.ones((NW, 1), jnp.float32)
    deg = jnp.dot(degw_ref[...], ones_col,
                  preferred_element_type=jnp.float32) + 1.0
    return lax.rsqrt(deg)


def _enc_body(x_ref, degw_ref, W1_ref, b1_ref, W2_ref, b2_ref, Wc1_ref, out_ref):
    h = jnp.maximum(jnp.dot(x_ref[...], W1_ref[...],
                            preferred_element_type=jnp.float32) + b1_ref[...], 0.0)
    h = jnp.dot(h, W2_ref[...], preferred_element_type=jnp.float32) + b2_ref[...]
    hw = jnp.dot(h, Wc1_ref[...], preferred_element_type=jnp.float32)
    out_ref[...] = hw * _dinv_block(degw_ref)


def _mid_body(accp_ref, hws_ref, degw_ref, bc1_ref, Wc2_ref, out_ref):
    dinv = _dinv_block(degw_ref)
    out1 = (accp_ref[0] + accp_ref[1] + hws_ref[...]) * dinv + bc1_ref[...]
    out_ref[...] = jnp.dot(out1, Wc2_ref[...],
                           preferred_element_type=jnp.float32) * dinv


def _fin_body(accp_ref, hws_ref, degw_ref, bc2_ref, batch_ref, Wcls_ref, bcls_ref,
              h_ref, logits_ref, seg_acc, cnt_acc):
    i = pl.program_id(0)
    dinv = _dinv_block(degw_ref)
    h = (accp_ref[0] + accp_ref[1] + hws_ref[...]) * dinv + bc2_ref[...]
    h_ref[...] = h
    gids = lax.broadcasted_iota(jnp.int32, (N_GRAPHS, R), 0)
    onehot = (batch_ref[0] == gids).astype(jnp.float32)
    part = jnp.dot(onehot, h, preferred_element_type=jnp.float32)
    cnt = jnp.broadcast_to(jnp.sum(onehot, axis=1, keepdims=True), (N_GRAPHS, D))

    @pl.when(i == 0)
    def _():
        seg_acc[...] = part
        cnt_acc[...] = cnt

    @pl.when(i > 0)
    def _():
        seg_acc[...] += part
        cnt_acc[...] += cnt

    @pl.when(i == pl.num_programs(0) - 1)
    def _():
        hg = seg_acc[...] / jnp.maximum(cnt_acc[...], 1.0)
        logits_ref[...] = jnp.dot(hg, Wcls_ref[...],
                                  preferred_element_type=jnp.float32) + bcls_ref[...]


def _row_spec(width):
    return pl.BlockSpec((R, width), lambda i: (i, 0))


def _degw_spec():
    return pl.BlockSpec((R, NW), lambda i: (i, 0))


def _part_spec(width):
    return pl.BlockSpec((NC, R, width), lambda i: (0, i, 0))


def _full_spec(shape):
    nd = len(shape)
    return pl.BlockSpec(shape, lambda i, _nd=nd: (0,) * _nd)


_enc_call = pl.pallas_call(
    _enc_body,
    grid=(GRID,),
    in_specs=[
        _row_spec(D), _degw_spec(),
        _full_spec((D, D)), _full_spec((1, D)),
        _full_spec((D, D)), _full_spec((1, D)),
        _full_spec((D, D)),
    ],
    out_specs=_row_spec(D),
    out_shape=jax.ShapeDtypeStruct((N_NODES, D), jnp.float32),
)

_mid_call = pl.pallas_call(
    _mid_body,
    grid=(GRID,),
    in_specs=[
        _part_spec(D), _row_spec(D), _degw_spec(),
        _full_spec((1, D)), _full_spec((D, D)),
    ],
    out_specs=_row_spec(D),
    out_shape=jax.ShapeDtypeStruct((N_NODES, D), jnp.float32),
)

_fin_call = pl.pallas_call(
    _fin_body,
    grid=(GRID,),
    in_specs=[
        _part_spec(D), _row_spec(D), _degw_spec(),
        _full_spec((1, D)),
        pl.BlockSpec((1, 1, R), lambda i: (i, 0, 0)),
        _full_spec((D, D)), _full_spec((1, D)),
    ],
    out_specs=[_row_spec(D), _full_spec((N_GRAPHS, D))],
    out_shape=[
        jax.ShapeDtypeStruct((N_NODES, D), jnp.float32),
        jax.ShapeDtypeStruct((N_GRAPHS, D), jnp.float32),
    ],
    scratch_shapes=[
        pltpu.VMEM((N_GRAPHS, D), jnp.float32),
        pltpu.VMEM((N_GRAPHS, D), jnp.float32),
    ],
)


def kernel(x, edge_index, batch, W1, b1, W2, b2, Wc1, bc1, Wc2, bc2, Wcls, bcls):
    src = edge_index[0]
    dst = edge_index[1]
    src3 = src.reshape(NW, EW // K, K)
    dst3 = dst.reshape(NW, EW // K, K)
    zeros128 = jnp.zeros((CH, D), jnp.float32)

    _deg_kernel, _agg_kernel = _sc_kernels()
    degw = _deg_kernel(dst).T

    hw1s = _enc_call(x, degw, W1, b1.reshape(1, D), W2, b2.reshape(1, D), Wc1)
    acc1 = _agg_kernel(hw1s, src3, dst3, zeros128)
    hw2s = _mid_call(acc1, hw1s, degw, bc1.reshape(1, D), Wc2)
    acc2 = _agg_kernel(hw2s, src3, dst3, zeros128)

    Wcls_p = jnp.pad(Wcls, ((0, 0), (0, D - D_OUT)))
    bcls_p = jnp.pad(bcls, (0, D - D_OUT)).reshape(1, D)
    batch_r = batch.reshape(GRID, 1, R)
    h_out, logits_p = _fin_call(acc2, hw2s, degw, bc2.reshape(1, D),
                                batch_r, Wcls_p, bcls_p)
    return (logits_p[:, :D_OUT], h_out)


# speedup vs baseline: 2.4430x; 1.1407x over previous
import functools

import jax
import jax.numpy as jnp
from jax import lax
from jax.experimental import pallas as pl
from jax.experimental.pallas import tpu as pltpu
from jax.experimental.pallas import tpu_sc as plsc

N_NODES = 10000
N_EDGES = 320000
D = 128
N_GRAPHS = 64
D_OUT = 100

NC = 2
NS = 16
NW = NC * NS
EW = N_EDGES // NW
K = 80
CH = 624
TAIL = N_NODES - NS * CH

R = 2000
GRID = N_NODES // R


@functools.cache
def _sc_kernels():
    mesh = plsc.VectorSubcoreMesh(core_axis_name="c", subcore_axis_name="s",
                                  num_cores=NC, num_subcores=NS)

    @functools.partial(
        pl.kernel,
        out_type=jax.ShapeDtypeStruct((NW, N_NODES), jnp.float32),
        mesh=mesh,
        compiler_params=pltpu.CompilerParams(needs_layout_passes=False),
        scratch_types=[
            pltpu.VMEM((K,), jnp.int32),
            pltpu.VMEM((16,), jnp.float32),
            pltpu.VMEM((N_NODES,), jnp.float32),
        ],
    )
    def deg_kernel(dst_hbm, out_hbm, didx, ones_v, tab):
        c = lax.axis_index("c")
        s = lax.axis_index("s")
        wid = s * NC + c
        ones_v[...] = jnp.full((16,), 1.0, jnp.float32)
        zv = ones_v[...] * 0.0

        def zbody(i, carry):
            tab[pl.ds(i * 16, 16)] = zv
            return carry

        lax.fori_loop(0, N_NODES // 16, zbody, 0)
        base = wid * EW

        def body(i, carry):
            pltpu.sync_copy(dst_hbm.at[pl.ds(base + i * K, K)], didx)
            ov = ones_v[...]
            for j in range(K // 16):
                dvec = didx[pl.ds(j * 16, 16)]
                plsc.addupdate_scatter(tab, [dvec], ov)
            return carry

        lax.fori_loop(0, EW // K, body, 0)
        pltpu.sync_copy(tab, out_hbm.at[wid])

    NCHUNK = EW // K

    @functools.partial(
        pl.kernel,
        out_type=jax.ShapeDtypeStruct((NC, N_NODES, D), jnp.float32),
        mesh=mesh,
        scratch_types=[
            pltpu.VMEM((K,), jnp.int32), pltpu.VMEM((K,), jnp.int32),
            pltpu.VMEM((K,), jnp.int32), pltpu.VMEM((K,), jnp.int32),
            pltpu.VMEM((K,), jnp.int32), pltpu.VMEM((K,), jnp.int32),
            pltpu.VMEM((K,), jnp.int32), pltpu.VMEM((K,), jnp.int32),
            pltpu.VMEM((K, D), jnp.float32), pltpu.VMEM((K, D), jnp.float32),
            pltpu.VMEM((K, D), jnp.float32), pltpu.VMEM((K, D), jnp.float32),
            pltpu.VMEM_SHARED((N_NODES, D), jnp.float32),
            pltpu.SemaphoreType.DMA, pltpu.SemaphoreType.DMA,
            pltpu.SemaphoreType.DMA, pltpu.SemaphoreType.DMA,
            pltpu.SemaphoreType.DMA, pltpu.SemaphoreType.DMA,
            pltpu.SemaphoreType.DMA, pltpu.SemaphoreType.DMA,
            pltpu.SemaphoreType.DMA, pltpu.SemaphoreType.DMA,
            pltpu.SemaphoreType.DMA, pltpu.SemaphoreType.DMA,
        ],
    )
    def agg_kernel(hw_hbm, src_hbm, dst_hbm, zeros_hbm, out_hbm,
                   si0, di0, si1, di1, si2, di2, si3, di3,
                   r0, r1, r2, r3, acc,
                   m0, m1, m2, m3, s0, s1, s2, s3, j0, j1, j2, j3):
        c = lax.axis_index("c")
        s = lax.axis_index("s")
        wid = s * NC + c
        pltpu.sync_copy(zeros_hbm.at[pl.ds(0, CH)], acc.at[pl.ds(s * CH, CH)])

        @pl.when(s == NS - 1)
        def _():
            pltpu.sync_copy(zeros_hbm.at[pl.ds(0, TAIL)],
                            acc.at[pl.ds(NS * CH, TAIL)])

        plsc.subcore_barrier()

        base = wid * EW
        dummy = hw_hbm.at[pl.ds(0, K)]
        idummy = src_hbm.at[pl.ds(0, K)]
        sis = (si0, si1, si2, si3)
        dis = (di0, di1, di2, di3)
        rows = (r0, r1, r2, r3)
        ms = (m0, m1, m2, m3)
        ss = (s0, s1, s2, s3)
        js = (j0, j1, j2, j3)

        for k in range(4):
            pltpu.sync_copy(src_hbm.at[pl.ds(base + k * K, K)], sis[k])
            pltpu.sync_copy(dst_hbm.at[pl.ds(base + k * K, K)], dis[k])
            pltpu.async_copy(hw_hbm.at[sis[k]], rows[k], ms[k])

        def body(i, carry):
            n = 4 * i
            for k in range(4):
                pltpu.make_async_copy(dummy, rows[k], ms[k]).wait()
                pltpu.async_copy(rows[k], acc.at[dis[k]], ss[k], add=True)
            for k in range(4):
                pltpu.make_async_copy(dummy, rows[k], ss[k]).wait()
                off = base + (n + 4 + k) * K
                pltpu.async_copy(src_hbm.at[pl.ds(off, K)], sis[k], js[k])
                pltpu.async_copy(dst_hbm.at[pl.ds(off, K)], dis[k], js[k])
                pltpu.make_async_copy(idummy, sis[k], js[k]).wait()
                pltpu.make_async_copy(idummy, dis[k], js[k]).wait()
                pltpu.async_copy(hw_hbm.at[sis[k]], rows[k], ms[k])
            return carry

        lax.fori_loop(0, (NCHUNK - 5) // 4, body, 0)
        for k in range(4):
            pltpu.make_async_copy(dummy, rows[k], ms[k]).wait()
            pltpu.async_copy(rows[k], acc.at[dis[k]], ss[k], add=True)
        pltpu.make_async_copy(dummy, rows[0], ss[0]).wait()
        pltpu.sync_copy(src_hbm.at[pl.ds(base + (NCHUNK - 1) * K, K)], si0)
        pltpu.sync_copy(dst_hbm.at[pl.ds(base + (NCHUNK - 1) * K, K)], di0)
        pltpu.async_copy(hw_hbm.at[si0], r0, m0)
        pltpu.make_async_copy(dummy, r0, m0).wait()
        pltpu.async_copy(r0, acc.at[di0], s0, add=True)
        for k in range(4):
            pltpu.make_async_copy(dummy, rows[k], ss[k]).wait()

        plsc.subcore_barrier()
        pltpu.sync_copy(acc.at[pl.ds(s * CH, CH)],
                        out_hbm.at[c, pl.ds(s * CH, CH)])

        @pl.when(s == NS - 1)
        def _():
            pltpu.sync_copy(acc.at[pl.ds(NS * CH, TAIL)],
                            out_hbm.at[c, pl.ds(NS * CH, TAIL)])

    return deg_kernel, agg_kernel


def _dinv_block(degw_ref):
    ones_col = jnp.ones((NW, 1), jnp.float32)
    deg = jnp.dot(degw_ref[...], ones_col,
                  preferred_element_type=jnp.float32) + 1.0
    return lax.rsqrt(deg)


def _enc_body(x_ref, degw_ref, W1_ref, b1_ref, W2_ref, b2_ref, Wc1_ref, out_ref):
    h = jnp.maximum(jnp.dot(x_ref[...], W1_ref[...],
                            preferred_element_type=jnp.float32) + b1_ref[...], 0.0)
    h = jnp.dot(h, W2_ref[...], preferred_element_type=jnp.float32) + b2_ref[...]
    hw = jnp.dot(h, Wc1_ref[...], preferred_element_type=jnp.float32)
    out_ref[...] = hw * _dinv_block(degw_ref)


def _mid_body(accp_ref, hws_ref, degw_ref, bc1_ref, Wc2_ref, out_ref):
    dinv = _dinv_block(degw_ref)
    out1 = (accp_ref[0] + accp_ref[1] + hws_ref[...]) * dinv + bc1_ref[...]
    out_ref[...] = jnp.dot(out1, Wc2_ref[...],
                           preferred_element_type=jnp.float32) * dinv


def _fin_body(accp_ref, hws_ref, degw_ref, bc2_ref, batch_ref, Wcls_ref, bcls_ref,
              h_ref, logits_ref, seg_acc, cnt_acc):
    i = pl.program_id(0)
    dinv = _dinv_block(degw_ref)
    h = (accp_ref[0] + accp_ref[1] + hws_ref[...]) * dinv + bc2_ref[...]
    h_ref[...] = h
    gids = lax.broadcasted_iota(jnp.int32, (N_GRAPHS, R), 0)
    onehot = (batch_ref[0] == gids).astype(jnp.float32)
    part = jnp.dot(onehot, h, preferred_element_type=jnp.float32)
    cnt = jnp.broadcast_to(jnp.sum(onehot, axis=1, keepdims=True), (N_GRAPHS, D))

    @pl.when(i == 0)
    def _():
        seg_acc[...] = part
        cnt_acc[...] = cnt

    @pl.when(i > 0)
    def _():
        seg_acc[...] += part
        cnt_acc[...] += cnt

    @pl.when(i == pl.num_programs(0) - 1)
    def _():
        hg = seg_acc[...] / jnp.maximum(cnt_acc[...], 1.0)
        logits_ref[...] = jnp.dot(hg, Wcls_ref[...],
                                  preferred_element_type=jnp.float32) + bcls_ref[...]


def _row_spec(width):
    return pl.BlockSpec((R, width), lambda i: (i, 0))


def _degw_spec():
    return pl.BlockSpec((R, NW), lambda i: (i, 0))


def _part_spec(width):
    return pl.BlockSpec((NC, R, width), lambda i: (0, i, 0))


def _full_spec(shape):
    nd = len(shape)
    return pl.BlockSpec(shape, lambda i, _nd=nd: (0,) * _nd)


_enc_call = pl.pallas_call(
    _enc_body,
    grid=(GRID,),
    in_specs=[
        _row_spec(D), _degw_spec(),
        _full_spec((D, D)), _full_spec((1, D)),
        _full_spec((D, D)), _full_spec((1, D)),
        _full_spec((D, D)),
    ],
    out_specs=_row_spec(D),
    out_shape=jax.ShapeDtypeStruct((N_NODES, D), jnp.float32),
)

_mid_call = pl.pallas_call(
    _mid_body,
    grid=(GRID,),
    in_specs=[
        _part_spec(D), _row_spec(D), _degw_spec(),
        _full_spec((1, D)), _full_spec((D, D)),
    ],
    out_specs=_row_spec(D),
    out_shape=jax.ShapeDtypeStruct((N_NODES, D), jnp.float32),
)

_fin_call = pl.pallas_call(
    _fin_body,
    grid=(GRID,),
    in_specs=[
        _part_spec(D), _row_spec(D), _degw_spec(),
        _full_spec((1, D)),
        pl.BlockSpec((1, 1, R), lambda i: (i, 0, 0)),
        _full_spec((D, D)), _full_spec((1, D)),
    ],
    out_specs=[_row_spec(D), _full_spec((N_GRAPHS, D))],
    out_shape=[
        jax.ShapeDtypeStruct((N_NODES, D), jnp.float32),
        jax.ShapeDtypeStruct((N_GRAPHS, D), jnp.float32),
    ],
    scratch_shapes=[
        pltpu.VMEM((N_GRAPHS, D), jnp.float32),
        pltpu.VMEM((N_GRAPHS, D), jnp.float32),
    ],
)


def kernel(x, edge_index, batch, W1, b1, W2, b2, Wc1, bc1, Wc2, bc2, Wcls, bcls):
    src = edge_index[0]
    dst = edge_index[1]

    zeros128 = jnp.zeros((CH, D), jnp.float32)

    _deg_kernel, _agg_kernel = _sc_kernels()
    degw = _deg_kernel(dst).T

    hw1s = _enc_call(x, degw, W1, b1.reshape(1, D), W2, b2.reshape(1, D), Wc1)
    acc1 = _agg_kernel(hw1s, src, dst, zeros128)
    hw2s = _mid_call(acc1, hw1s, degw, bc1.reshape(1, D), Wc2)
    acc2 = _agg_kernel(hw2s, src, dst, zeros128)

    Wcls_p = jnp.pad(Wcls, ((0, 0), (0, D - D_OUT)))
    bcls_p = jnp.pad(bcls, (0, D - D_OUT)).reshape(1, D)
    batch_r = batch.reshape(GRID, 1, R)
    h_out, logits_p = _fin_call(acc2, hw2s, degw, bc2.reshape(1, D),
                                batch_r, Wcls_p, bcls_p)
    return (logits_p[:, :D_OUT], h_out)


# speedup vs baseline: 2.6450x; 1.0827x over previous
import functools

import jax
import jax.numpy as jnp
from jax import lax
from jax.experimental import pallas as pl
from jax.experimental.pallas import tpu as pltpu
from jax.experimental.pallas import tpu_sc as plsc

N_NODES = 10000
N_EDGES = 320000
D = 128
N_GRAPHS = 64
D_OUT = 100

NC = 2
NS = 16
NW = NC * NS
EW = N_EDGES // NW
K = 80
CH = 624
TAIL = N_NODES - NS * CH

R = 2000
GRID = N_NODES // R


@functools.cache
def _sc_kernels():
    mesh = plsc.VectorSubcoreMesh(core_axis_name="c", subcore_axis_name="s",
                                  num_cores=NC, num_subcores=NS)

    @functools.partial(
        pl.kernel,
        out_type=jax.ShapeDtypeStruct((NW, N_NODES), jnp.float32),
        mesh=mesh,
        compiler_params=pltpu.CompilerParams(needs_layout_passes=False),
        scratch_types=[
            pltpu.VMEM((K,), jnp.int32),
            pltpu.VMEM((K,), jnp.int32),
            pltpu.VMEM((16,), jnp.float32),
            pltpu.VMEM((N_NODES,), jnp.float32),
            pltpu.SemaphoreType.DMA,
            pltpu.SemaphoreType.DMA,
        ],
    )
    def deg_kernel(dst_hbm, out_hbm, didx_a, didx_b, ones_v, tab, ja, jb):
        c = lax.axis_index("c")
        s = lax.axis_index("s")
        wid = s * NC + c
        ones_v[...] = jnp.full((16,), 1.0, jnp.float32)
        zv = ones_v[...] * 0.0

        def zbody(i, carry):
            tab[pl.ds(i * 16, 16)] = zv
            return carry

        lax.fori_loop(0, N_NODES // 16, zbody, 0)
        base = wid * EW
        idummy = dst_hbm.at[pl.ds(0, K)]
        NCHD = EW // K

        def scat(buf):
            ov = ones_v[...]
            for j in range(K // 16):
                dvec = buf[pl.ds(j * 16, 16)]
                plsc.addupdate_scatter(tab, [dvec], ov)

        pltpu.sync_copy(dst_hbm.at[pl.ds(base, K)], didx_a)
        pltpu.async_copy(dst_hbm.at[pl.ds(base + K, K)], didx_b, jb)

        def body(i, carry):
            n = 2 * i
            scat(didx_a)
            pltpu.async_copy(dst_hbm.at[pl.ds(base + (n + 2) * K, K)], didx_a, ja)
            pltpu.make_async_copy(idummy, didx_b, jb).wait()
            scat(didx_b)
            nn = jnp.minimum(n + 3, NCHD - 1)
            pltpu.async_copy(dst_hbm.at[pl.ds(base + nn * K, K)], didx_b, jb)
            pltpu.make_async_copy(idummy, didx_a, ja).wait()
            return carry

        lax.fori_loop(0, (NCHD - 1) // 2, body, 0)
        pltpu.make_async_copy(idummy, didx_b, jb).wait()
        scat(didx_a)
        pltpu.sync_copy(tab, out_hbm.at[wid])

    NCHUNK = EW // K

    @functools.partial(
        pl.kernel,
        out_type=jax.ShapeDtypeStruct((NC, N_NODES, D), jnp.float32),
        mesh=mesh,
        scratch_types=[
            pltpu.VMEM((K,), jnp.int32), pltpu.VMEM((K,), jnp.int32),
            pltpu.VMEM((K,), jnp.int32), pltpu.VMEM((K,), jnp.int32),
            pltpu.VMEM((K,), jnp.int32), pltpu.VMEM((K,), jnp.int32),
            pltpu.VMEM((K,), jnp.int32), pltpu.VMEM((K,), jnp.int32),
            pltpu.VMEM((K, D), jnp.float32), pltpu.VMEM((K, D), jnp.float32),
            pltpu.VMEM((K, D), jnp.float32), pltpu.VMEM((K, D), jnp.float32),
            pltpu.VMEM_SHARED((N_NODES, D), jnp.float32),
            pltpu.SemaphoreType.DMA, pltpu.SemaphoreType.DMA,
            pltpu.SemaphoreType.DMA, pltpu.SemaphoreType.DMA,
            pltpu.SemaphoreType.DMA, pltpu.SemaphoreType.DMA,
            pltpu.SemaphoreType.DMA, pltpu.SemaphoreType.DMA,
            pltpu.SemaphoreType.DMA, pltpu.SemaphoreType.DMA,
            pltpu.SemaphoreType.DMA, pltpu.SemaphoreType.DMA,
        ],
    )
    def agg_kernel(hw_hbm, src_hbm, dst_hbm, zeros_hbm, out_hbm,
                   si0, di0, si1, di1, si2, di2, si3, di3,
                   r0, r1, r2, r3, acc,
                   m0, m1, m2, m3, s0, s1, s2, s3, j0, j1, j2, j3):
        c = lax.axis_index("c")
        s = lax.axis_index("s")
        wid = s * NC + c
        pltpu.sync_copy(zeros_hbm.at[pl.ds(0, CH)], acc.at[pl.ds(s * CH, CH)])

        @pl.when(s == NS - 1)
        def _():
            pltpu.sync_copy(zeros_hbm.at[pl.ds(0, TAIL)],
                            acc.at[pl.ds(NS * CH, TAIL)])

        plsc.subcore_barrier()

        base = wid * EW
        dummy = hw_hbm.at[pl.ds(0, K)]
        idummy = src_hbm.at[pl.ds(0, K)]
        sis = (si0, si1, si2, si3)
        dis = (di0, di1, di2, di3)
        rows = (r0, r1, r2, r3)
        ms = (m0, m1, m2, m3)
        ss = (s0, s1, s2, s3)
        js = (j0, j1, j2, j3)

        for k in range(4):
            pltpu.sync_copy(src_hbm.at[pl.ds(base + k * K, K)], sis[k])
            pltpu.sync_copy(dst_hbm.at[pl.ds(base + k * K, K)], dis[k])
            pltpu.async_copy(hw_hbm.at[sis[k]], rows[k], ms[k])

        def body(i, carry):
            n = 4 * i
            for k in range(4):
                pltpu.make_async_copy(dummy, rows[k], ms[k]).wait()
                pltpu.async_copy(rows[k], acc.at[dis[k]], ss[k], add=True)
            for k in range(4):
                pltpu.make_async_copy(dummy, rows[k], ss[k]).wait()
                off = base + (n + 4 + k) * K
                pltpu.async_copy(src_hbm.at[pl.ds(off, K)], sis[k], js[k])
                pltpu.async_copy(dst_hbm.at[pl.ds(off, K)], dis[k], js[k])
                pltpu.make_async_copy(idummy, sis[k], js[k]).wait()
                pltpu.make_async_copy(idummy, dis[k], js[k]).wait()
                pltpu.async_copy(hw_hbm.at[sis[k]], rows[k], ms[k])
            return carry

        lax.fori_loop(0, (NCHUNK - 5) // 4, body, 0)
        for k in range(4):
            pltpu.make_async_copy(dummy, rows[k], ms[k]).wait()
            pltpu.async_copy(rows[k], acc.at[dis[k]], ss[k], add=True)
        pltpu.make_async_copy(dummy, rows[0], ss[0]).wait()
        pltpu.sync_copy(src_hbm.at[pl.ds(base + (NCHUNK - 1) * K, K)], si0)
        pltpu.sync_copy(dst_hbm.at[pl.ds(base + (NCHUNK - 1) * K, K)], di0)
        pltpu.async_copy(hw_hbm.at[si0], r0, m0)
        pltpu.make_async_copy(dummy, r0, m0).wait()
        pltpu.async_copy(r0, acc.at[di0], s0, add=True)
        for k in range(4):
            pltpu.make_async_copy(dummy, rows[k], ss[k]).wait()

        plsc.subcore_barrier()
        pltpu.sync_copy(acc.at[pl.ds(s * CH, CH)],
                        out_hbm.at[c, pl.ds(s * CH, CH)])

        @pl.when(s == NS - 1)
        def _():
            pltpu.sync_copy(acc.at[pl.ds(NS * CH, TAIL)],
                            out_hbm.at[c, pl.ds(NS * CH, TAIL)])

    return deg_kernel, agg_kernel


def _dinv_block(degw_ref):
    ones_col = jnp.ones((NW, 1), jnp.float32)
    deg = jnp.dot(degw_ref[...], ones_col,
                  preferred_element_type=jnp.float32) + 1.0
    return lax.rsqrt(deg)


def _enc_body(x_ref, W1_ref, b1_ref, W2_ref, b2_ref, Wc1_ref, out_ref):
    h = jnp.maximum(jnp.dot(x_ref[...], W1_ref[...],
                            preferred_element_type=jnp.float32) + b1_ref[...], 0.0)
    h = jnp.dot(h, W2_ref[...], preferred_element_type=jnp.float32) + b2_ref[...]
    out_ref[...] = jnp.dot(h, Wc1_ref[...], preferred_element_type=jnp.float32)


def _scale_body(hw_ref, degw_ref, out_ref):
    out_ref[...] = hw_ref[...] * _dinv_block(degw_ref)


def _mid_body(accp_ref, hws_ref, degw_ref, bc1_ref, Wc2_ref, out_ref):
    dinv = _dinv_block(degw_ref)
    out1 = (accp_ref[0] + accp_ref[1] + hws_ref[...]) * dinv + bc1_ref[...]
    out_ref[...] = jnp.dot(out1, Wc2_ref[...],
                           preferred_element_type=jnp.float32) * dinv


def _fin_body(accp_ref, hws_ref, degw_ref, bc2_ref, batch_ref, Wcls_ref, bcls_ref,
              h_ref, logits_ref, seg_acc, cnt_acc):
    i = pl.program_id(0)
    dinv = _dinv_block(degw_ref)
    h = (accp_ref[0] + accp_ref[1] + hws_ref[...]) * dinv + bc2_ref[...]
    h_ref[...] = h
    gids = lax.broadcasted_iota(jnp.int32, (N_GRAPHS, R), 0)
    onehot = (batch_ref[0] == gids).astype(jnp.float32)
    part = jnp.dot(onehot, h, preferred_element_type=jnp.float32)
    cnt = jnp.broadcast_to(jnp.sum(onehot, axis=1, keepdims=True), (N_GRAPHS, D))

    @pl.when(i == 0)
    def _():
        seg_acc[...] = part
        cnt_acc[...] = cnt

    @pl.when(i > 0)
    def _():
        seg_acc[...] += part
        cnt_acc[...] += cnt

    @pl.when(i == pl.num_programs(0) - 1)
    def _():
        hg = seg_acc[...] / jnp.maximum(cnt_acc[...], 1.0)
        logits_ref[...] = jnp.dot(hg, Wcls_ref[...],
                                  preferred_element_type=jnp.float32) + bcls_ref[...]


def _row_spec(width):
    return pl.BlockSpec((R, width), lambda i: (i, 0))


def _degw_spec():
    return pl.BlockSpec((R, NW), lambda i: (i, 0))


def _part_spec(width):
    return pl.BlockSpec((NC, R, width), lambda i: (0, i, 0))


def _full_spec(shape):
    nd = len(shape)
    return pl.BlockSpec(shape, lambda i, _nd=nd: (0,) * _nd)


_enc_call = pl.pallas_call(
    _enc_body,
    grid=(GRID,),
    in_specs=[
        _row_spec(D),
        _full_spec((D, D)), _full_spec((1, D)),
        _full_spec((D, D)), _full_spec((1, D)),
        _full_spec((D, D)),
    ],
    out_specs=_row_spec(D),
    out_shape=jax.ShapeDtypeStruct((N_NODES, D), jnp.float32),
)

_scale_call = pl.pallas_call(
    _scale_body,
    grid=(GRID,),
    in_specs=[_row_spec(D), _degw_spec()],
    out_specs=_row_spec(D),
    out_shape=jax.ShapeDtypeStruct((N_NODES, D), jnp.float32),
)

_mid_call = pl.pallas_call(
    _mid_body,
    grid=(GRID,),
    in_specs=[
        _part_spec(D), _row_spec(D), _degw_spec(),
        _full_spec((1, D)), _full_spec((D, D)),
    ],
    out_specs=_row_spec(D),
    out_shape=jax.ShapeDtypeStruct((N_NODES, D), jnp.float32),
)

_fin_call = pl.pallas_call(
    _fin_body,
    grid=(GRID,),
    in_specs=[
        _part_spec(D), _row_spec(D), _degw_spec(),
        _full_spec((1, D)),
        pl.BlockSpec((1, 1, R), lambda i: (i, 0, 0)),
        _full_spec((D, D)), _full_spec((1, D)),
    ],
    out_specs=[_row_spec(D), _full_spec((N_GRAPHS, D))],
    out_shape=[
        jax.ShapeDtypeStruct((N_NODES, D), jnp.float32),
        jax.ShapeDtypeStruct((N_GRAPHS, D), jnp.float32),
    ],
    scratch_shapes=[
        pltpu.VMEM((N_GRAPHS, D), jnp.float32),
        pltpu.VMEM((N_GRAPHS, D), jnp.float32),
    ],
)


def kernel(x, edge_index, batch, W1, b1, W2, b2, Wc1, bc1, Wc2, bc2, Wcls, bcls):
    src = edge_index[0]
    dst = edge_index[1]

    zeros128 = jnp.zeros((CH, D), jnp.float32)

    _deg_kernel, _agg_kernel = _sc_kernels()
    degw = _deg_kernel(dst).T

    hw1 = _enc_call(x, W1, b1.reshape(1, D), W2, b2.reshape(1, D), Wc1)
    hw1s = _scale_call(hw1, degw)
    acc1 = _agg_kernel(hw1s, src, dst, zeros128)
    hw2s = _mid_call(acc1, hw1s, degw, bc1.reshape(1, D), Wc2)
    acc2 = _agg_kernel(hw2s, src, dst, zeros128)

    Wcls_p = jnp.pad(Wcls, ((0, 0), (0, D - D_OUT)))
    bcls_p = jnp.pad(bcls, (0, D - D_OUT)).reshape(1, D)
    batch_r = batch.reshape(GRID, 1, R)
    h_out, logits_p = _fin_call(acc2, hw2s, degw, bc2.reshape(1, D),
                                batch_r, Wcls_p, bcls_p)
    return (logits_p[:, :D_OUT], h_out)
